# Initial kernel scaffold; baseline (speedup 1.0000x reference)
import jax
import jax.numpy as jnp
from jax.experimental import pallas as pl


def kernel(t, nlc, lut):
    raise NotImplementedError("write your pallas kernel here")


# speedup vs baseline: 1.6383x; 1.6383x over previous
import jax
import jax.numpy as jnp
from jax.experimental import pallas as pl
from jax.experimental.pallas import tpu as pltpu

N_LUTS = 1024
LUT_DIM = 33
T = 1024
FLAT = 3 * LUT_DIM * LUT_DIM * LUT_DIM
ROWS, COLS = 99, 1089


def _acc_body(u_ref, w_ref, row_ref, out_ref):
    i = pl.program_id(0)

    @pl.when(i == 0)
    def _init():
        out_ref[...] = jnp.zeros_like(out_ref)

    w = w_ref[i]

    @pl.when(w != 0.0)
    def _acc():
        out_ref[...] += w * row_ref[0]


def kernel(t, nlc, lut):
    lut3 = lut.reshape(N_LUTS, ROWS, COLS)
    grid_spec = pltpu.PrefetchScalarGridSpec(
        num_scalar_prefetch=2,
        grid=(T,),
        in_specs=[
            pl.BlockSpec((1, ROWS, COLS), lambda i, u, w: (u[i], 0, 0)),
        ],
        out_specs=pl.BlockSpec((ROWS, COLS), lambda i, u, w: (0, 0)),
    )
    out = pl.pallas_call(
        _acc_body,
        grid_spec=grid_spec,
        out_shape=jax.ShapeDtypeStruct((ROWS, COLS), jnp.float32),
    )(t.astype(jnp.int32), nlc, lut3)
    return out.reshape(3, LUT_DIM, LUT_DIM, LUT_DIM)


# speedup vs baseline: 13.3066x; 8.1224x over previous
import functools

import jax
import jax.numpy as jnp
from jax import lax
from jax.experimental import pallas as pl
from jax.experimental.pallas import tpu as pltpu
from jax.experimental.pallas import tpu_sc as plsc

N_LUTS = 1024
LUT_DIM = 33
T = 1024
P = 3 * LUT_DIM * LUT_DIM
PB = 33
L = 16


def _build_w_sc(t, nlc):
    mesh = plsc.VectorSubcoreMesh(core_axis_name="c", subcore_axis_name="s")

    @functools.partial(
        pl.kernel,
        mesh=mesh,
        out_type=jax.ShapeDtypeStruct((N_LUTS,), jnp.float32),
        compiler_params=pltpu.CompilerParams(needs_layout_passes=False),
        scratch_types=[
            pltpu.VMEM((L * N_LUTS,), jnp.float32),
            pltpu.VMEM((T,), jnp.int32),
            pltpu.VMEM((T,), jnp.float32),
            pltpu.VMEM((N_LUTS,), jnp.float32),
        ],
    )
    def k(t_hbm, nlc_hbm, w_hbm, hist_v, t_v, nlc_v, w_v):
        wid = lax.axis_index("s") * 2 + lax.axis_index("c")

        @pl.when(wid == 0)
        def _():
            pltpu.sync_copy(t_hbm, t_v)
            pltpu.sync_copy(nlc_hbm, nlc_v)
            lanes = lax.iota(jnp.int32, L)

            def zero_body(j, carry):
                hist_v[pl.ds(j * L, L)] = jnp.zeros((L,), jnp.float32)
                return carry

            lax.fori_loop(0, L * N_LUTS // L, zero_body, 0)

            def scat_body(j, carry):
                idx = t_v[pl.ds(j * L, L)]
                val = nlc_v[pl.ds(j * L, L)]
                plsc.addupdate_scatter(hist_v, [lanes * N_LUTS + idx], val)
                return carry

            lax.fori_loop(0, T // L, scat_body, 0)

            def red_body(j, carry):
                acc = hist_v[pl.ds(j * L, L)]
                for r in range(1, L):
                    acc = acc + hist_v[pl.ds(r * N_LUTS + j * L, L)]
                w_v[pl.ds(j * L, L)] = acc
                return carry

            lax.fori_loop(0, N_LUTS // L, red_body, 0)
            pltpu.sync_copy(w_v, w_hbm)

    return k(t, nlc)


def _matvec_body(w_ref, blk_ref, out_ref):
    out_ref[0] = jnp.sum(blk_ref[...] * w_ref[...], axis=-1)


def kernel(t, nlc, lut):
    w = _build_w_sc(t.astype(jnp.int32), nlc)
    lut3 = jnp.transpose(lut, (1, 2, 3, 4, 0)).reshape(P, LUT_DIM, N_LUTS)
    out2 = pl.pallas_call(
        _matvec_body,
        grid=(P // PB,),
        in_specs=[
            pl.BlockSpec((N_LUTS,), lambda i: (0,)),
            pl.BlockSpec((PB, LUT_DIM, N_LUTS), lambda i: (i, 0, 0)),
        ],
        out_specs=pl.BlockSpec((1, PB, LUT_DIM), lambda i: (i, 0, 0)),
        out_shape=jax.ShapeDtypeStruct((P // PB, PB, LUT_DIM), jnp.float32),
    )(w, lut3)
    return out2.reshape(3, LUT_DIM, LUT_DIM, LUT_DIM)


# speedup vs baseline: 13.6181x; 1.0234x over previous
import functools

import jax
import jax.numpy as jnp
from jax import lax
from jax.experimental import pallas as pl
from jax.experimental.pallas import tpu as pltpu
from jax.experimental.pallas import tpu_sc as plsc

N_LUTS = 1024
LUT_DIM = 33
T = 1024
P = 3 * LUT_DIM * LUT_DIM
PB = 33
L = 16


def _build_w_sc(t, nlc):
    mesh = plsc.VectorSubcoreMesh(core_axis_name="c", subcore_axis_name="s")

    @functools.partial(
        pl.kernel,
        mesh=mesh,
        out_type=jax.ShapeDtypeStruct((N_LUTS,), jnp.float32),
        compiler_params=pltpu.CompilerParams(needs_layout_passes=False),
        scratch_types=[
            pltpu.VMEM((L * N_LUTS,), jnp.float32),
            pltpu.VMEM((T,), jnp.int32),
            pltpu.VMEM((T,), jnp.float32),
            pltpu.VMEM((N_LUTS,), jnp.float32),
        ],
    )
    def k(t_hbm, nlc_hbm, w_hbm, hist_v, t_v, nlc_v, w_v):
        wid = lax.axis_index("s") * 2 + lax.axis_index("c")

        @pl.when(wid == 0)
        def _():
            pltpu.sync_copy(t_hbm, t_v)
            pltpu.sync_copy(nlc_hbm, nlc_v)
            lanes = lax.iota(jnp.int32, L)

            def zero_body(j, carry):
                hist_v[pl.ds(j * L, L)] = jnp.zeros((L,), jnp.float32)
                return carry

            lax.fori_loop(0, L * N_LUTS // L, zero_body, 0)

            def scat_body(j, carry):
                idx = t_v[pl.ds(j * L, L)]
                val = nlc_v[pl.ds(j * L, L)]
                plsc.addupdate_scatter(hist_v, [lanes * N_LUTS + idx], val)
                return carry

            lax.fori_loop(0, T // L, scat_body, 0)

            def red_body(j, carry):
                acc = hist_v[pl.ds(j * L, L)]
                for r in range(1, L):
                    acc = acc + hist_v[pl.ds(r * N_LUTS + j * L, L)]
                w_v[pl.ds(j * L, L)] = acc
                return carry

            lax.fori_loop(0, N_LUTS // L, red_body, 0)
            pltpu.sync_copy(w_v, w_hbm)

    return k(t, nlc)


def _matvec_body(w_ref, blk0_ref, blk1_ref, blk2_ref, out_ref):
    out_ref[0] = jnp.sum(blk0_ref[...] * w_ref[...], axis=-1)
    out_ref[1] = jnp.sum(blk1_ref[...] * w_ref[...], axis=-1)
    out_ref[2] = jnp.sum(blk2_ref[...] * w_ref[...], axis=-1)


def kernel(t, nlc, lut):
    w = _build_w_sc(t.astype(jnp.int32), nlc)
    lut3 = jnp.transpose(lut, (1, 2, 3, 4, 0)).reshape(P, LUT_DIM, N_LUTS)
    out2 = pl.pallas_call(
        _matvec_body,
        grid=(P // PB // 3,),
        in_specs=[
            pl.BlockSpec((N_LUTS,), lambda i: (0,)),
            pl.BlockSpec((PB, LUT_DIM, N_LUTS), lambda i: (3 * i, 0, 0)),
            pl.BlockSpec((PB, LUT_DIM, N_LUTS), lambda i: (3 * i + 1, 0, 0)),
            pl.BlockSpec((PB, LUT_DIM, N_LUTS), lambda i: (3 * i + 2, 0, 0)),
        ],
        out_specs=pl.BlockSpec((3, PB, LUT_DIM), lambda i: (i, 0, 0)),
        out_shape=jax.ShapeDtypeStruct((P // PB, PB, LUT_DIM), jnp.float32),
    )(w, lut3, lut3, lut3)
    return out2.reshape(3, LUT_DIM, LUT_DIM, LUT_DIM)
